# baseline (speedup 1.0000x reference)
import functools

import jax
import jax.numpy as jnp
from jax import lax
from jax.experimental import pallas as pl
from jax.experimental.pallas import tpu as pltpu
from jax.experimental.pallas import tpu_sc as plsc

Q = 1024
N = 1000
LANES = 16
NPAD = 1008
NCHUNKS = NPAD // LANES
NC = 2
NS = 16
NW = NC * NS
ROWS_PER_W = Q // NW


def _sc_partials(rel_pad, par_pad):
    mesh = plsc.VectorSubcoreMesh(core_axis_name="c", subcore_axis_name="s")

    @functools.partial(
        pl.kernel,
        mesh=mesh,
        compiler_params=pltpu.CompilerParams(use_tc_tiling_on_sc=False,
                                              needs_layout_passes=False),
        out_type=jax.ShapeDtypeStruct((NW * LANES,), jnp.float32),
        scratch_types=[
            pltpu.VMEM((ROWS_PER_W, NPAD), jnp.float32),
            pltpu.VMEM((NPAD,), jnp.int32),
            pltpu.VMEM((LANES,), jnp.float32),
        ],
    )
    def k(rel_hbm, par_hbm, out_hbm, rows_v, par_v, acc_v):
        wid = lax.axis_index("s") * NC + lax.axis_index("c")
        base = wid * ROWS_PER_W
        pltpu.sync_copy(par_hbm, par_v)
        pltpu.sync_copy(rel_hbm.at[pl.ds(base, ROWS_PER_W)], rows_v)

        lane_f = lax.iota(jnp.int32, LANES).astype(jnp.float32)

        def chunk_body(c, acc):
            col0 = c * LANES
            parv = par_v[pl.ds(col0, LANES)]
            wv = 1.0 / (lane_f + (col0 + 1).astype(jnp.float32))
            s = jnp.zeros((LANES,), jnp.float32)
            for r in range(ROWS_PER_W):
                rv = rows_v[r, pl.ds(col0, LANES)]
                gv = plsc.load_gather(
                    rows_v, [jnp.full((LANES,), r, jnp.int32), parv])
                s = s + rv * gv
            return acc + wv * s

        acc = lax.fori_loop(0, NCHUNKS, chunk_body,
                            jnp.zeros((LANES,), jnp.float32))
        acc_v[...] = acc
        pltpu.sync_copy(acc_v, out_hbm.at[pl.ds(wid * LANES, LANES)])

    return k(rel_pad, par_pad)


def _tc_epilogue(partials, par_f):
    def body(p_ref, par_ref, map_ref, cum_ref):
        total = jnp.sum(p_ref[...])
        map_ref[...] = jnp.full((1, 1), total / (N * Q), jnp.float32)
        flat = (lax.broadcasted_iota(jnp.int32, (8, 128), 0) * 128
                + lax.broadcasted_iota(jnp.int32, (8, 128), 1))
        cum_val = jnp.sum(jnp.where(flat < 100, par_ref[...], 0.0)) / 100.0
        cum_ref[...] = jnp.full((8, 128), cum_val, jnp.float32)

    return pl.pallas_call(
        body,
        out_shape=(
            jax.ShapeDtypeStruct((1, 1), jnp.float32),
            jax.ShapeDtypeStruct((8, 128), jnp.float32),
        ),
    )(partials, par_f)


def kernel(relevances, precision_at_ranks):
    rel = relevances.astype(jnp.float32)
    par = precision_at_ranks.astype(jnp.int32)
    rel_pad = jnp.pad(rel, ((0, 0), (0, NPAD - N)))
    par_pad = jnp.pad(par, (0, NPAD - N))
    partials = _sc_partials(rel_pad, par_pad).reshape(NW, LANES)
    par_f = jnp.pad(par.astype(jnp.float32), (0, Q - N)).reshape(8, 128)
    map_out, cum_out = _tc_epilogue(partials, par_f)
    return (map_out[0, 0], cum_out.reshape(Q))


# speedup vs baseline: 1.1827x; 1.1827x over previous
import functools

import jax
import jax.numpy as jnp
from jax import lax
from jax.experimental import pallas as pl
from jax.experimental.pallas import tpu as pltpu
from jax.experimental.pallas import tpu_sc as plsc

Q = 1024
N = 1000
LANES = 16
NCHUNKS = (N + LANES - 1) // LANES
NC = 2
NS = 16
NW = NC * NS
ROWS_PER_W = Q // NW
NOUT = NW * LANES + 2 * LANES


def _sc_partials(rel, par):
    mesh = plsc.VectorSubcoreMesh(core_axis_name="c", subcore_axis_name="s")

    @functools.partial(
        pl.kernel,
        mesh=mesh,
        compiler_params=pltpu.CompilerParams(use_tc_tiling_on_sc=False,
                                             needs_layout_passes=False),
        out_type=jax.ShapeDtypeStruct((NOUT,), jnp.float32),
        scratch_types=[
            pltpu.VMEM((ROWS_PER_W, N), jnp.float32),
            pltpu.VMEM((N,), jnp.int32),
            pltpu.VMEM((LANES,), jnp.float32),
            pltpu.VMEM((2 * LANES,), jnp.float32),
        ],
    )
    def k(rel_hbm, par_hbm, out_hbm, rows_v, par_v, acc_v, cum_v):
        wid = lax.axis_index("s") * NC + lax.axis_index("c")
        base = wid * ROWS_PER_W
        pltpu.sync_copy(par_hbm, par_v)
        pltpu.sync_copy(rel_hbm.at[pl.ds(base, ROWS_PER_W)], rows_v)

        lane = lax.iota(jnp.int32, LANES)
        lane_f = lane.astype(jnp.float32)

        def chunk_body(c, acc):
            col0 = jnp.minimum(c * LANES, N - LANES)
            parv = par_v[pl.ds(col0, LANES)]
            colf = lane_f + col0.astype(jnp.float32)
            wv = jnp.where(lane + col0 >= c * LANES, 1.0 / (colf + 1.0), 0.0)
            s = jnp.zeros((LANES,), jnp.float32)
            for r in range(ROWS_PER_W):
                rv = rows_v[r, pl.ds(col0, LANES)]
                gv = plsc.load_gather(
                    rows_v, [jnp.full((LANES,), r, jnp.int32), parv])
                s = s + rv * gv
            return acc + wv * s

        acc = lax.fori_loop(0, NCHUNKS, chunk_body,
                            jnp.zeros((LANES,), jnp.float32))
        acc_v[...] = acc
        pltpu.sync_copy(acc_v, out_hbm.at[pl.ds(wid * LANES, LANES)])

        @pl.when(wid == 0)
        def _():
            cums = jnp.zeros((LANES,), jnp.float32)
            for c in range(6):
                cums = cums + par_v[pl.ds(c * LANES, LANES)].astype(jnp.float32)
            tail = par_v[pl.ds(6 * LANES, LANES)].astype(jnp.float32)
            cums = cums + jnp.where(lane < 4, tail, 0.0)
            cum_v[pl.ds(0, LANES)] = cums
            cum_v[pl.ds(LANES, LANES)] = jnp.zeros((LANES,), jnp.float32)
            pltpu.sync_copy(cum_v, out_hbm.at[pl.ds(NW * LANES, 2 * LANES)])

    return k(rel, par)


def _tc_epilogue(partials):
    def body(p_ref, map_ref, cum_ref):
        total = jnp.sum(p_ref[pl.ds(0, NW * LANES)])
        cums = jnp.sum(p_ref[pl.ds(NW * LANES, 2 * LANES)])
        map_ref[...] = jnp.full((1,), total / (N * Q), jnp.float32)
        cum_ref[...] = jnp.full((Q,), cums / 100.0, jnp.float32)

    return pl.pallas_call(
        body,
        out_shape=(
            jax.ShapeDtypeStruct((1,), jnp.float32),
            jax.ShapeDtypeStruct((Q,), jnp.float32),
        ),
    )(partials)


def kernel(relevances, precision_at_ranks):
    rel = relevances.astype(jnp.float32)
    par = precision_at_ranks.astype(jnp.int32)
    partials = _sc_partials(rel, par)
    map_out, cum_out = _tc_epilogue(partials)
    return (map_out[0], cum_out)


# speedup vs baseline: 1.1841x; 1.0012x over previous
import functools

import jax
import jax.numpy as jnp
from jax import lax
from jax.experimental import pallas as pl
from jax.experimental.pallas import tpu as pltpu
from jax.experimental.pallas import tpu_sc as plsc

Q = 1024
N = 1000
LANES = 16
NCHUNKS = (N + LANES - 1) // LANES
NC = 2
NS = 16
NW = NC * NS
ROWS_PER_W = Q // NW
BLK = ROWS_PER_W * N
NOUT = NW * LANES + 2 * LANES


def _sc_partials(rel_flat, par):
    mesh = plsc.VectorSubcoreMesh(core_axis_name="c", subcore_axis_name="s")

    @functools.partial(
        pl.kernel,
        mesh=mesh,
        compiler_params=pltpu.CompilerParams(use_tc_tiling_on_sc=False,
                                             needs_layout_passes=False),
        out_type=jax.ShapeDtypeStruct((NOUT,), jnp.float32),
        scratch_types=[
            pltpu.VMEM((BLK,), jnp.float32),
            pltpu.VMEM((N,), jnp.int32),
            pltpu.VMEM((LANES,), jnp.float32),
            pltpu.VMEM((2 * LANES,), jnp.float32),
        ],
    )
    def k(rel_hbm, par_hbm, out_hbm, rows_v, par_v, acc_v, cum_v):
        wid = lax.axis_index("s") * NC + lax.axis_index("c")
        pltpu.sync_copy(par_hbm, par_v)
        pltpu.sync_copy(rel_hbm.at[pl.ds(wid * BLK, BLK)], rows_v)

        lane = lax.iota(jnp.int32, LANES)
        lane_f = lane.astype(jnp.float32)

        def chunk_body(c, acc):
            col0 = jnp.minimum(c * LANES, N - LANES)
            parv = par_v[pl.ds(col0, LANES)]
            colf = lane_f + col0.astype(jnp.float32)
            wv = jnp.where(lane + col0 >= c * LANES, 1.0 / (colf + 1.0), 0.0)
            s = jnp.zeros((LANES,), jnp.float32)
            for r in range(ROWS_PER_W):
                rv = rows_v[pl.ds(r * N + col0, LANES)]
                gv = plsc.load_gather(rows_v, [parv + r * N])
                s = s + rv * gv
            return acc + wv * s

        acc = lax.fori_loop(0, NCHUNKS, chunk_body,
                            jnp.zeros((LANES,), jnp.float32))
        acc_v[...] = acc
        pltpu.sync_copy(acc_v, out_hbm.at[pl.ds(wid * LANES, LANES)])

        @pl.when(wid == 0)
        def _():
            cums = jnp.zeros((LANES,), jnp.float32)
            for c in range(6):
                cums = cums + par_v[pl.ds(c * LANES, LANES)].astype(jnp.float32)
            tail = par_v[pl.ds(6 * LANES, LANES)].astype(jnp.float32)
            cums = cums + jnp.where(lane < 4, tail, 0.0)
            cum_v[pl.ds(0, LANES)] = cums
            cum_v[pl.ds(LANES, LANES)] = jnp.zeros((LANES,), jnp.float32)
            pltpu.sync_copy(cum_v, out_hbm.at[pl.ds(NW * LANES, 2 * LANES)])

    return k(rel_flat, par)


def _tc_epilogue(partials):
    def body(p_ref, map_ref, cum_ref):
        total = jnp.sum(p_ref[pl.ds(0, NW * LANES)])
        cums = jnp.sum(p_ref[pl.ds(NW * LANES, 2 * LANES)])
        map_ref[...] = jnp.full((1,), total / (N * Q), jnp.float32)
        cum_ref[...] = jnp.full((Q,), cums / 100.0, jnp.float32)

    return pl.pallas_call(
        body,
        out_shape=(
            jax.ShapeDtypeStruct((1,), jnp.float32),
            jax.ShapeDtypeStruct((Q,), jnp.float32),
        ),
    )(partials)


def kernel(relevances, precision_at_ranks):
    rel_flat = relevances.astype(jnp.float32).reshape(Q * N)
    par = precision_at_ranks.astype(jnp.int32)
    partials = _sc_partials(rel_flat, par)
    map_out, cum_out = _tc_epilogue(partials)
    return (map_out[0], cum_out)


# speedup vs baseline: 1.2875x; 1.0873x over previous
import functools

import jax
import jax.numpy as jnp
from jax import lax
from jax.experimental import pallas as pl
from jax.experimental.pallas import tpu as pltpu
from jax.experimental.pallas import tpu_sc as plsc

Q = 1024
N = 1000
LANES = 16
NCHUNKS = (N + LANES - 1) // LANES
NC = 2
NS = 16
NW = NC * NS
ROWS_PER_W = Q // NW
NOUT = NW * LANES + 2 * LANES


def _sc_partials(rel, par):
    mesh = plsc.VectorSubcoreMesh(core_axis_name="c", subcore_axis_name="s")

    @functools.partial(
        pl.kernel,
        mesh=mesh,
        compiler_params=pltpu.CompilerParams(use_tc_tiling_on_sc=True,
                                             needs_layout_passes=False),
        out_type=jax.ShapeDtypeStruct((NOUT,), jnp.float32),
        scratch_types=[
            pltpu.VMEM((ROWS_PER_W, N), jnp.float32),
            pltpu.VMEM((N,), jnp.int32),
            pltpu.VMEM((LANES,), jnp.float32),
            pltpu.VMEM((2 * LANES,), jnp.float32),
        ],
    )
    def k(rel_hbm, par_hbm, out_hbm, rows_v, par_v, acc_v, cum_v):
        wid = lax.axis_index("s") * NC + lax.axis_index("c")
        base = wid * ROWS_PER_W
        pltpu.sync_copy(par_hbm, par_v)
        pltpu.sync_copy(rel_hbm.at[pl.ds(base, ROWS_PER_W)], rows_v)

        lane = lax.iota(jnp.int32, LANES)
        lane_f = lane.astype(jnp.float32)

        def chunk_body(c, acc):
            col0 = jnp.minimum(c * LANES, N - LANES)
            parv = par_v[pl.ds(col0, LANES)]
            colf = lane_f + col0.astype(jnp.float32)
            wv = jnp.where(lane + col0 >= c * LANES, 1.0 / (colf + 1.0), 0.0)
            s = jnp.zeros((LANES,), jnp.float32)
            for r in range(ROWS_PER_W):
                rv = rows_v[r, pl.ds(col0, LANES)]
                gv = plsc.load_gather(
                    rows_v, [jnp.full((LANES,), r, jnp.int32), parv])
                s = s + rv * gv
            return acc + wv * s

        acc = lax.fori_loop(0, NCHUNKS, chunk_body,
                            jnp.zeros((LANES,), jnp.float32))
        acc_v[...] = acc
        pltpu.sync_copy(acc_v, out_hbm.at[pl.ds(wid * LANES, LANES)])

        @pl.when(wid == 0)
        def _():
            cums = jnp.zeros((LANES,), jnp.float32)
            for c in range(6):
                cums = cums + par_v[pl.ds(c * LANES, LANES)].astype(jnp.float32)
            tail = par_v[pl.ds(6 * LANES, LANES)].astype(jnp.float32)
            cums = cums + jnp.where(lane < 4, tail, 0.0)
            cum_v[pl.ds(0, LANES)] = cums
            cum_v[pl.ds(LANES, LANES)] = jnp.zeros((LANES,), jnp.float32)
            pltpu.sync_copy(cum_v, out_hbm.at[pl.ds(NW * LANES, 2 * LANES)])

    return k(rel, par)


def _tc_epilogue(partials):
    def body(p_ref, map_ref, cum_ref):
        total = jnp.sum(p_ref[pl.ds(0, NW * LANES)])
        cums = jnp.sum(p_ref[pl.ds(NW * LANES, 2 * LANES)])
        map_ref[...] = jnp.full((1,), total / (N * Q), jnp.float32)
        cum_ref[...] = jnp.full((Q,), cums / 100.0, jnp.float32)

    return pl.pallas_call(
        body,
        out_shape=(
            jax.ShapeDtypeStruct((1,), jnp.float32),
            jax.ShapeDtypeStruct((Q,), jnp.float32),
        ),
    )(partials)


def kernel(relevances, precision_at_ranks):
    rel = relevances.astype(jnp.float32)
    par = precision_at_ranks.astype(jnp.int32)
    partials = _sc_partials(rel, par)
    map_out, cum_out = _tc_epilogue(partials)
    return (map_out[0], cum_out)


# speedup vs baseline: 1.4176x; 1.1011x over previous
import functools

import jax
import jax.numpy as jnp
from jax import lax
from jax.experimental import pallas as pl
from jax.experimental.pallas import tpu as pltpu
from jax.experimental.pallas import tpu_sc as plsc

Q = 1024
N = 1000
LANES = 16
QCHUNKS = Q // LANES
NC = 2
NS = 16
NW = NC * NS
J_PER_W = 32
NOUT = NW * LANES + 2 * LANES


def _sc_partials(rt, par):
    mesh = plsc.VectorSubcoreMesh(core_axis_name="c", subcore_axis_name="s")

    @functools.partial(
        pl.kernel,
        mesh=mesh,
        compiler_params=pltpu.CompilerParams(use_tc_tiling_on_sc=True,
                                             needs_layout_passes=False),
        out_type=jax.ShapeDtypeStruct((NOUT,), jnp.float32),
        scratch_types=[
            pltpu.VMEM((J_PER_W, Q), jnp.float32),
            pltpu.VMEM((J_PER_W, Q), jnp.float32),
            pltpu.VMEM((J_PER_W,), jnp.int32),
            pltpu.VMEM((7 * LANES,), jnp.int32),
            pltpu.VMEM((J_PER_W + LANES,), jnp.float32),
            pltpu.VMEM((LANES,), jnp.float32),
            pltpu.VMEM((2 * LANES,), jnp.float32),
            pltpu.SemaphoreType.DMA,
        ],
    )
    def k(rt_hbm, par_hbm, out_hbm, myrows_v, grows_v, par_v, par100_v,
          w_v, acc_v, cum_v, sem):
        wid = lax.axis_index("s") * NC + lax.axis_index("c")
        lo = jnp.minimum(wid * J_PER_W, N - J_PER_W)
        pltpu.sync_copy(par_hbm.at[pl.ds(lo, J_PER_W)], par_v)
        gather = pltpu.async_copy(rt_hbm.at[par_v], grows_v, sem)
        pltpu.sync_copy(rt_hbm.at[pl.ds(lo, J_PER_W)], myrows_v)

        lane = lax.iota(jnp.int32, LANES)
        for h in range(J_PER_W // LANES + 1):
            jg = lane + (lo + h * LANES)
            wvec = jnp.where((jg >= wid * J_PER_W) & (jg < lo + J_PER_W),
                             1.0 / (jg + 1).astype(jnp.float32), 0.0)
            w_v[pl.ds(h * LANES, LANES)] = wvec
        gather.wait()

        def j_body(j, acc):
            wj = w_v[pl.ds(j, LANES)][0]
            t = jnp.zeros((LANES,), jnp.float32)
            for c in range(QCHUNKS):
                a = myrows_v[j, pl.ds(c * LANES, LANES)]
                b = grows_v[j, pl.ds(c * LANES, LANES)]
                t = t + a * b
            return acc + wj * t

        acc = lax.fori_loop(0, J_PER_W, j_body,
                            jnp.zeros((LANES,), jnp.float32))
        acc_v[...] = acc
        pltpu.sync_copy(acc_v, out_hbm.at[pl.ds(wid * LANES, LANES)])

        @pl.when(wid == 0)
        def _():
            pltpu.sync_copy(par_hbm.at[pl.ds(0, 7 * LANES)], par100_v)
            cums = jnp.zeros((LANES,), jnp.float32)
            for c in range(6):
                cums = cums + par100_v[pl.ds(c * LANES, LANES)].astype(
                    jnp.float32)
            tail = par100_v[pl.ds(6 * LANES, LANES)].astype(jnp.float32)
            cums = cums + jnp.where(lane < 4, tail, 0.0)
            cum_v[pl.ds(0, LANES)] = cums
            cum_v[pl.ds(LANES, LANES)] = jnp.zeros((LANES,), jnp.float32)
            pltpu.sync_copy(cum_v, out_hbm.at[pl.ds(NW * LANES, 2 * LANES)])

    return k(rt, par)


def _tc_epilogue(partials):
    def body(p_ref, map_ref, cum_ref):
        total = jnp.sum(p_ref[pl.ds(0, NW * LANES)])
        cums = jnp.sum(p_ref[pl.ds(NW * LANES, 2 * LANES)])
        map_ref[...] = jnp.full((1,), total / (N * Q), jnp.float32)
        cum_ref[...] = jnp.full((Q,), cums / 100.0, jnp.float32)

    return pl.pallas_call(
        body,
        out_shape=(
            jax.ShapeDtypeStruct((1,), jnp.float32),
            jax.ShapeDtypeStruct((Q,), jnp.float32),
        ),
    )(partials)


def kernel(relevances, precision_at_ranks):
    rt = relevances.astype(jnp.float32).T
    par = precision_at_ranks.astype(jnp.int32)
    partials = _sc_partials(rt, par)
    map_out, cum_out = _tc_epilogue(partials)
    return (map_out[0], cum_out)
